# Initial kernel scaffold; baseline (speedup 1.0000x reference)
import jax
import jax.numpy as jnp
from jax.experimental import pallas as pl


def kernel(target, context, target_table, context_table):
    raise NotImplementedError("write your pallas kernel here")


# speedup vs baseline: 1.0645x; 1.0645x over previous
import functools

import jax
import jax.numpy as jnp
from jax import lax
from jax.experimental import pallas as pl
from jax.experimental.pallas import tpu as pltpu
from jax.experimental.pallas import tpu_sc as plsc

_VOCAB = 1000000
_EMBED = 128
_BATCH = 16384
_L = 16
_NC = 2
_NS = 16
_NW = _NC * _NS
_B_PER_W = _BATCH // _NW
_CHUNK = 256
_N_CHUNKS = _B_PER_W // _CHUNK
_GROUPS = _CHUNK // _L


def _body(tidx_hbm, cidx_hbm, tt_hbm, ct_hbm, out_hbm,
          tidx_v, cidx_v, trows_v, crows_v, out_v, sem_t, sem_c):
    wid = lax.axis_index("s") * _NC + lax.axis_index("c")
    base = wid * _B_PER_W

    for ch in range(_N_CHUNKS):
        cb = ch * _CHUNK
        pltpu.sync_copy(tidx_hbm.at[pl.ds(base + cb, _CHUNK)], tidx_v)
        pltpu.sync_copy(cidx_hbm.at[pl.ds(base + cb, _CHUNK)], cidx_v)
        t_cp = pltpu.async_copy(tt_hbm.at[tidx_v], trows_v, sem_t)
        c_cp = pltpu.async_copy(ct_hbm.at[cidx_v], crows_v, sem_c)
        t_cp.wait()
        c_cp.wait()

        def group(g, carry):
            res = jnp.zeros((_L,), jnp.float32)
            lane = lax.iota(jnp.int32, _L)
            for r in range(_L):
                row = g * _L + r
                acc = trows_v[row, pl.ds(0, _L)] * crows_v[row, pl.ds(0, _L)]
                for i in range(1, _EMBED // _L):
                    acc = acc + (trows_v[row, pl.ds(i * _L, _L)] *
                                 crows_v[row, pl.ds(i * _L, _L)])
                for sh in (8, 4, 2, 1):
                    acc = acc + acc.at[lane ^ sh].get(
                        mode="promise_in_bounds")
                res = jnp.where(lane == r, acc, res)
            out_v[pl.ds(cb + g * _L, _L)] = res
            return carry

        lax.fori_loop(0, _GROUPS, group, 0)

    pltpu.sync_copy(out_v, out_hbm.at[pl.ds(base, _B_PER_W)])


@jax.jit
def kernel(target, context, target_table, context_table):
    mesh = plsc.VectorSubcoreMesh(core_axis_name="c", subcore_axis_name="s")
    run = pl.kernel(
        _body,
        mesh=mesh,
        out_type=jax.ShapeDtypeStruct((_BATCH,), jnp.float32),
        scratch_types=[
            pltpu.VMEM((_CHUNK,), jnp.int32),
            pltpu.VMEM((_CHUNK,), jnp.int32),
            pltpu.VMEM((_CHUNK, _EMBED), jnp.float32),
            pltpu.VMEM((_CHUNK, _EMBED), jnp.float32),
            pltpu.VMEM((_B_PER_W,), jnp.float32),
            pltpu.SemaphoreType.DMA,
            pltpu.SemaphoreType.DMA,
        ],
    )
    flat = run(target.astype(jnp.int32), context.astype(jnp.int32),
               target_table, context_table)
    return flat.reshape(_BATCH, 1)
